# baseline (speedup 1.0000x reference)
import jax
import jax.numpy as jnp
from jax import lax
from jax.experimental import pallas as pl
from jax.experimental.pallas import tpu as pltpu
from jax.experimental.pallas import tpu_sc as plsc

MEM_TOKENS = 50
DIM = 64
GRP = 8

NC = 2
NS = 16
NW = NC * NS

CHUNK = 64


def _sc_gather(idx_hbm, table_hbm, out_hbm, idx_v, obuf, dsem, wsem):
    n = idx_hbm.shape[0]
    per_w = n // NW
    nchunk = per_w // CHUNK
    wid = lax.axis_index("s") * NC + lax.axis_index("c")
    base = wid * per_w
    pltpu.sync_copy(idx_hbm.at[pl.ds(base, per_w)], idx_v)

    iota16 = lax.iota(jnp.int32, 16)
    zeros16 = jnp.zeros((16,), jnp.float32)

    def fire(c, b):
        for g in range(CHUNK // 16):
            idx16 = idx_v[pl.ds(c * CHUNK + g * 16, 16)]
            for l in range(16):
                s = idx16[l]
                gi = lax.shift_right_logical(s, 3)
                ri = lax.bitwise_and(s, 7)
                pltpu.make_async_copy(
                    table_hbm.at[gi, pl.ds(ri, 1)],
                    obuf.at[b, pl.ds(g * 16 + l, 1)],
                    dsem.at[b]).start()

    def drain(b):
        pltpu.make_async_copy(
            out_hbm.at[pl.ds(0, CHUNK)], obuf.at[b], dsem.at[b]).wait()

    def mask_pass(c, b):
        nz_total = jnp.sum(jnp.where(idx_v[pl.ds(c * CHUNK, 16)] == 0, 1, 0))
        for g in range(1, CHUNK // 16):
            nz_total = nz_total + jnp.sum(jnp.where(
                idx_v[pl.ds(c * CHUNK + g * 16, 16)] == 0, 1, 0))

        @pl.when(nz_total > 0)
        def _():
            for g in range(CHUNK // 16):
                idx16 = idx_v[pl.ds(c * CHUNK + g * 16, 16)]
                z = idx16 == 0
                tok16 = iota16 + (g * 16)
                for c0 in range(DIM):
                    col = jnp.full((16,), c0, jnp.int32)
                    plsc.store_scatter(obuf.at[b], [tok16, col], zeros16,
                                       mask=z)

    def write(c, b):
        return pltpu.make_async_copy(
            obuf.at[b], out_hbm.at[pl.ds(base + c * CHUNK, CHUNK)],
            wsem.at[b])

    fire(0, 0)

    def body(i, _):
        for b in range(2):
            c = i * 2 + b
            nb = 1 - b

            @pl.when(c + 1 < nchunk)
            def _():
                @pl.when(c >= 1)
                def _():
                    write(c - 1, nb).wait()
                fire(c + 1, nb)

            drain(b)
            mask_pass(c, b)
            write(c, b).start()
        return 0

    lax.fori_loop(0, nchunk // 2, body, 0)

    write(nchunk - 2, 0).wait()
    write(nchunk - 1, 1).wait()


def _tc_matmul(h_ref, wt_ref, o_ref):
    nb = o_ref.shape[0]
    acc = jnp.dot(h_ref[...], wt_ref[...], preferred_element_type=jnp.float32)
    o_ref[...] = acc.reshape(nb, MEM_TOKENS, DIM)


def kernel(x, emb_table, W_proj):
    B, L = x.shape
    n = B * MEM_TOKENS
    idx = x[:, -MEM_TOKENS:].reshape(n).astype(jnp.int32)
    table3 = emb_table.reshape(emb_table.shape[0] // GRP, GRP, DIM)
    wt = W_proj.T

    mesh = plsc.VectorSubcoreMesh(core_axis_name="c", subcore_axis_name="s")
    K = 2
    ns = n // K
    per_w = ns // NW
    gather = pl.kernel(
        _sc_gather,
        mesh=mesh,
        out_type=jax.ShapeDtypeStruct((ns, DIM), jnp.float32),
        scratch_types=[
            pltpu.VMEM((per_w,), jnp.int32),
            pltpu.VMEM((2, CHUNK, DIM), jnp.float32),
            pltpu.SemaphoreType.DMA((2,)),
            pltpu.SemaphoreType.DMA((2,)),
        ],
        compiler_params=pltpu.CompilerParams(needs_layout_passes=False),
    )

    BB = 64
    Bs = B // K
    matmul = pl.pallas_call(
        _tc_matmul,
        grid=(Bs // BB,),
        in_specs=[
            pl.BlockSpec((BB * MEM_TOKENS, DIM), lambda i: (i, 0)),
            pl.BlockSpec((DIM, DIM), lambda i: (0, 0)),
        ],
        out_specs=pl.BlockSpec((BB, MEM_TOKENS, DIM), lambda i: (i, 0, 0)),
        out_shape=jax.ShapeDtypeStruct((Bs, MEM_TOKENS, DIM), jnp.float32),
    )

    outs = []
    for k in range(K):
        rows = gather(lax.slice(idx, (k * ns,), ((k + 1) * ns,)), table3)
        outs.append(matmul(rows, wt))

    return jnp.concatenate(outs, axis=0)


# speedup vs baseline: 1.1557x; 1.1557x over previous
import jax
import jax.numpy as jnp
from jax import lax
from jax.experimental import pallas as pl
from jax.experimental.pallas import tpu as pltpu
from jax.experimental.pallas import tpu_sc as plsc

MEM_TOKENS = 50
DIM = 64
GRP = 8

NC = 2
NS = 16
NW = NC * NS

K = 2
MT = MEM_TOKENS // K
CHUNK = MT

WINS = [(0, range(16)), (MT - 16, range(32 - MT, 16))]


def _sc_gather(idx_hbm, table_hbm, out_hbm, idx_v, obuf, dsem, wsem):
    n = idx_hbm.shape[0]
    per_w = n // NW
    nchunk = per_w // CHUNK
    wid = lax.axis_index("s") * NC + lax.axis_index("c")
    base = wid * per_w
    pltpu.sync_copy(idx_hbm.at[pl.ds(base, per_w)], idx_v)

    iota16 = lax.iota(jnp.int32, 16)
    zeros16 = jnp.zeros((16,), jnp.float32)

    def fire(c, b):
        for off, lanes in WINS:
            idx16 = idx_v[pl.ds(c * CHUNK + off, 16)]
            for l in lanes:
                s = idx16[l]
                gi = lax.shift_right_logical(s, 3)
                ri = lax.bitwise_and(s, 7)
                pltpu.make_async_copy(
                    table_hbm.at[gi, pl.ds(ri, 1)],
                    obuf.at[b, off + l],
                    dsem.at[b]).start()

    def drain(b):
        pltpu.make_async_copy(
            out_hbm.at[pl.ds(0, CHUNK), 0, pl.ds(0, 1)], obuf.at[b],
            dsem.at[b]).wait()

    def mask_pass(c, b):
        nz = jnp.zeros((16,), jnp.int32)
        for off, lanes in WINS:
            idx16 = idx_v[pl.ds(c * CHUNK + off, 16)]
            if len(lanes) < 16:
                idx16 = jnp.where(iota16 >= lanes[0], idx16, 1)
            nz = nz + jnp.where(idx16 == 0, 1, 0)

        @pl.when(jnp.sum(nz) > 0)
        def _():
            zrow = jnp.zeros((16,), jnp.int32)
            for off, lanes in WINS:
                idx16 = idx_v[pl.ds(c * CHUNK + off, 16)]
                z = idx16 == 0
                if len(lanes) < 16:
                    z = z & (iota16 >= lanes[0])
                tok16 = iota16 + off
                for c0 in range(DIM):
                    col = jnp.full((16,), c0, jnp.int32)
                    plsc.store_scatter(obuf.at[b], [tok16, zrow, col],
                                       zeros16, mask=z)

    def write(c, b):
        bb = base // CHUNK + c
        b8 = lax.shift_right_logical(bb, 3)
        br = lax.bitwise_and(bb, 7)
        return pltpu.make_async_copy(
            obuf.at[b],
            out_hbm.at[:, b8, pl.ds(br, 1)],
            wsem.at[b])

    fire(0, 0)

    def body(i, _):
        for b in range(2):
            c = i * 2 + b
            nb = 1 - b

            @pl.when(c + 1 < nchunk)
            def _():
                @pl.when(c >= 1)
                def _():
                    write(c - 1, nb).wait()
                fire(c + 1, nb)

            drain(b)
            mask_pass(c, b)
            write(c, b).start()
        return 0

    lax.fori_loop(0, nchunk // 2, body, 0)

    write(nchunk - 2, 0).wait()
    write(nchunk - 1, 1).wait()


def _tc_matmul(h_ref, wt_ref, *rest):
    o_ref = rest[-1]
    h = h_ref[0]
    acc = jnp.dot(h, wt_ref[...], preferred_element_type=jnp.float32)
    o_ref[0] = acc.T


def kernel(x, emb_table, W_proj):
    B, L = x.shape
    ns = B * MT
    table3 = emb_table.reshape(emb_table.shape[0] // GRP, GRP, DIM)
    wt = W_proj.T

    mesh = plsc.VectorSubcoreMesh(core_axis_name="c", subcore_axis_name="s")
    per_w = ns // NW
    gather = pl.kernel(
        _sc_gather,
        mesh=mesh,
        out_type=jax.ShapeDtypeStruct((MT, B // GRP, GRP, DIM), jnp.float32),
        scratch_types=[
            pltpu.VMEM((per_w,), jnp.int32),
            pltpu.VMEM((2, CHUNK, 1, DIM), jnp.float32),
            pltpu.SemaphoreType.DMA((2,)),
            pltpu.SemaphoreType.DMA((2,)),
        ],
        compiler_params=pltpu.CompilerParams(needs_layout_passes=False),
    )

    def matmul(k, rows, prev):
        in_specs = [
            pl.BlockSpec((1, B, DIM), lambda i: (i, 0, 0)),
            pl.BlockSpec((DIM, DIM), lambda i: (0, 0)),
        ]
        args = [rows, wt]
        aliases = {}
        if prev is not None:
            in_specs.append(
                pl.BlockSpec((1, DIM, B), lambda i, k=k: (i + k * MT, 0, 0)))
            args.append(prev)
            aliases = {2: 0}
        return pl.pallas_call(
            _tc_matmul,
            grid=(MT,),
            in_specs=in_specs,
            out_specs=pl.BlockSpec((1, DIM, B),
                                   lambda i, k=k: (i + k * MT, 0, 0)),
            out_shape=jax.ShapeDtypeStruct((MEM_TOKENS, DIM, B), jnp.float32),
            input_output_aliases=aliases,
        )(*args)

    t0 = L - MEM_TOKENS
    outT = None
    for k in range(K):
        idx_k = x[:, t0 + k * MT: t0 + (k + 1) * MT].reshape(ns)
        idx_k = idx_k.astype(jnp.int32)
        rows = gather(idx_k, table3)
        rows = rows.reshape(MT, B, DIM)
        outT = matmul(k, rows, outT)

    return outT.transpose(2, 0, 1)


# speedup vs baseline: 1.2548x; 1.0857x over previous
import jax
import jax.numpy as jnp
from jax import lax
from jax.experimental import pallas as pl
from jax.experimental.pallas import tpu as pltpu
from jax.experimental.pallas import tpu_sc as plsc

MEM_TOKENS = 50
DIM = 64
GRP = 8

NC = 2
NS = 16
NW = NC * NS

CHUNK = MEM_TOKENS


def _sc_gather(idx_hbm, table_hbm, out_hbm, idx_v, obuf, dsem, wsem):
    n = idx_hbm.shape[0]
    per_w = n // NW
    nchunk = per_w // CHUNK
    wid = lax.axis_index("s") * NC + lax.axis_index("c")
    base = wid * per_w
    pltpu.sync_copy(idx_hbm.at[pl.ds(base, per_w)], idx_v)

    iota16 = lax.iota(jnp.int32, 16)
    zeros16 = jnp.zeros((16,), jnp.float32)
    wins = [(0, range(16)), (16, range(16)), (32, range(16)),
            (34, range(14, 16))]

    def fire(c, b):
        for off, lanes in wins:
            idx16 = idx_v[pl.ds(c * CHUNK + off, 16)]
            for l in lanes:
                s = idx16[l]
                gi = lax.shift_right_logical(s, 3)
                ri = lax.bitwise_and(s, 7)
                pltpu.make_async_copy(
                    table_hbm.at[gi, pl.ds(ri, 1)],
                    obuf.at[b, off + l],
                    dsem.at[b]).start()

    def drain(b):
        pltpu.make_async_copy(
            out_hbm.at[pl.ds(0, CHUNK), 0, pl.ds(0, 1)], obuf.at[b],
            dsem.at[b]).wait()

    def mask_pass(c, b):
        nz = jnp.zeros((16,), jnp.int32)
        for off, lanes in wins:
            idx16 = idx_v[pl.ds(c * CHUNK + off, 16)]
            if len(lanes) < 16:
                idx16 = jnp.where(iota16 >= lanes[0], idx16, 1)
            nz = nz + jnp.where(idx16 == 0, 1, 0)

        @pl.when(jnp.sum(nz) > 0)
        def _():
            zrow = jnp.zeros((16,), jnp.int32)
            for off, lanes in wins:
                idx16 = idx_v[pl.ds(c * CHUNK + off, 16)]
                z = idx16 == 0
                if len(lanes) < 16:
                    z = z & (iota16 >= lanes[0])
                tok16 = iota16 + off
                for c0 in range(DIM):
                    col = jnp.full((16,), c0, jnp.int32)
                    plsc.store_scatter(obuf.at[b], [tok16, zrow, col],
                                       zeros16, mask=z)

    def write(c, b):
        bb = base // CHUNK + c
        b8 = lax.shift_right_logical(bb, 3)
        br = lax.bitwise_and(bb, 7)
        return pltpu.make_async_copy(
            obuf.at[b],
            out_hbm.at[:, b8, pl.ds(br, 1)],
            wsem.at[b])

    fire(0, 0)

    def body(i, _):
        for b in range(2):
            c = i * 2 + b
            nb = 1 - b

            @pl.when(c + 1 < nchunk)
            def _():
                @pl.when(c >= 1)
                def _():
                    write(c - 1, nb).wait()
                fire(c + 1, nb)

            drain(b)
            mask_pass(c, b)
            write(c, b).start()
        return 0

    lax.fori_loop(0, nchunk // 2, body, 0)

    write(nchunk - 2, 0).wait()
    write(nchunk - 1, 1).wait()


def _tc_matmul(h_ref, wt_ref, o_ref):
    h = h_ref[0]
    acc = jnp.dot(h, wt_ref[...], preferred_element_type=jnp.float32)
    o_ref[0] = acc.T


def kernel(x, emb_table, W_proj):
    B, L = x.shape
    n = B * MEM_TOKENS
    idx = x[:, -MEM_TOKENS:].reshape(n).astype(jnp.int32)
    table3 = emb_table.reshape(emb_table.shape[0] // GRP, GRP, DIM)

    mesh = plsc.VectorSubcoreMesh(core_axis_name="c", subcore_axis_name="s")
    per_w = n // NW
    gather = pl.kernel(
        _sc_gather,
        mesh=mesh,
        out_type=jax.ShapeDtypeStruct((MEM_TOKENS, B // GRP, GRP, DIM),
                                      jnp.float32),
        scratch_types=[
            pltpu.VMEM((per_w,), jnp.int32),
            pltpu.VMEM((2, CHUNK, 1, DIM), jnp.float32),
            pltpu.SemaphoreType.DMA((2,)),
            pltpu.SemaphoreType.DMA((2,)),
        ],
        compiler_params=pltpu.CompilerParams(needs_layout_passes=False),
    )
    rows = gather(idx, table3)
    rows = rows.reshape(MEM_TOKENS, B, DIM)

    outT = pl.pallas_call(
        _tc_matmul,
        grid=(MEM_TOKENS,),
        in_specs=[
            pl.BlockSpec((1, B, DIM), lambda i: (i, 0, 0)),
            pl.BlockSpec((DIM, DIM), lambda i: (0, 0)),
        ],
        out_specs=pl.BlockSpec((1, DIM, B), lambda i: (i, 0, 0)),
        out_shape=jax.ShapeDtypeStruct((MEM_TOKENS, DIM, B), jnp.float32),
    )(rows, W_proj.T)

    return outT.transpose(2, 0, 1)


# speedup vs baseline: 1.3102x; 1.0442x over previous
import jax
import jax.numpy as jnp
from jax import lax
from jax.experimental import pallas as pl
from jax.experimental.pallas import tpu as pltpu
from jax.experimental.pallas import tpu_sc as plsc

MEM_TOKENS = 50
DIM = 64
GRP = 8

NC = 2
NS = 16
NW = NC * NS

CHUNK = MEM_TOKENS


def _sc_gather(idx_hbm, table_hbm, out_hbm, idx_v, obuf, dsem, wsem):
    n = idx_hbm.shape[0]
    per_w = n // NW
    nchunk = per_w // CHUNK
    wid = lax.axis_index("s") * NC + lax.axis_index("c")
    base = wid * per_w
    pltpu.sync_copy(idx_hbm.at[pl.ds(base, per_w)], idx_v)

    iota16 = lax.iota(jnp.int32, 16)
    zeros16 = jnp.zeros((16,), jnp.float32)
    wins = [(0, range(16)), (16, range(16)), (32, range(16)),
            (34, range(14, 16))]

    def fire(c, b):
        for off, lanes in wins:
            idx16 = idx_v[pl.ds(c * CHUNK + off, 16)]
            for l in lanes:
                s = idx16[l]
                gi = lax.shift_right_logical(s, 3)
                ri = lax.bitwise_and(s, 7)
                pltpu.make_async_copy(
                    table_hbm.at[gi, pl.ds(ri, 1)],
                    obuf.at[b, off + l],
                    dsem.at[b]).start()

    def drain(b):
        pltpu.make_async_copy(
            out_hbm.at[pl.ds(0, CHUNK), 0, pl.ds(0, 1)], obuf.at[b],
            dsem.at[b]).wait()

    def mask_pass(c, b):
        nz = jnp.zeros((16,), jnp.int32)
        for off, lanes in wins:
            idx16 = idx_v[pl.ds(c * CHUNK + off, 16)]
            if len(lanes) < 16:
                idx16 = jnp.where(iota16 >= lanes[0], idx16, 1)
            nz = nz + jnp.where(idx16 == 0, 1, 0)

        @pl.when(jnp.sum(nz) > 0)
        def _():
            zrow = jnp.zeros((16,), jnp.int32)
            for off, lanes in wins:
                idx16 = idx_v[pl.ds(c * CHUNK + off, 16)]
                z = idx16 == 0
                if len(lanes) < 16:
                    z = z & (iota16 >= lanes[0])
                tok16 = iota16 + off
                for c0 in range(DIM):
                    col = jnp.full((16,), c0, jnp.int32)
                    plsc.store_scatter(obuf.at[b], [tok16, zrow, col],
                                       zeros16, mask=z)

    def write(c, b):
        bb = base // CHUNK + c
        b8 = lax.shift_right_logical(bb, 3)
        br = lax.bitwise_and(bb, 7)
        return pltpu.make_async_copy(
            obuf.at[b],
            out_hbm.at[:, b8, pl.ds(br, 1)],
            wsem.at[b])

    fire(0, 0)

    def body(i, _):
        for b in range(2):
            c = i * 2 + b
            nb = 1 - b

            @pl.when(c + 1 < nchunk)
            def _():
                @pl.when(c >= 1)
                def _():
                    write(c - 1, nb).wait()
                fire(c + 1, nb)

            drain(b)
            mask_pass(c, b)
            write(c, b).start()
        return 0

    lax.fori_loop(0, nchunk // 2, body, 0)

    write(nchunk - 2, 0).wait()
    write(nchunk - 1, 1).wait()


def _tc_matmul(h_ref, wt_ref, o_ref):
    for t in range(h_ref.shape[0]):
        h = h_ref[t]
        acc = jnp.dot(h, wt_ref[...], preferred_element_type=jnp.float32)
        o_ref[t] = acc.T


def kernel(x, emb_table, W_proj):
    B, L = x.shape
    n = B * MEM_TOKENS
    idx = x[:, -MEM_TOKENS:].reshape(n).astype(jnp.int32)
    table3 = emb_table.reshape(emb_table.shape[0] // GRP, GRP, DIM)

    mesh = plsc.VectorSubcoreMesh(core_axis_name="c", subcore_axis_name="s")
    per_w = n // NW
    gather = pl.kernel(
        _sc_gather,
        mesh=mesh,
        out_type=jax.ShapeDtypeStruct((MEM_TOKENS, B // GRP, GRP, DIM),
                                      jnp.float32),
        scratch_types=[
            pltpu.VMEM((per_w,), jnp.int32),
            pltpu.VMEM((2, CHUNK, 1, DIM), jnp.float32),
            pltpu.SemaphoreType.DMA((2,)),
            pltpu.SemaphoreType.DMA((2,)),
        ],
        compiler_params=pltpu.CompilerParams(needs_layout_passes=False),
    )
    rows = gather(idx, table3)
    rows = rows.reshape(MEM_TOKENS, B, DIM)

    TB = 2
    outT = pl.pallas_call(
        _tc_matmul,
        grid=(MEM_TOKENS // TB,),
        in_specs=[
            pl.BlockSpec((TB, B, DIM), lambda i: (i, 0, 0)),
            pl.BlockSpec((DIM, DIM), lambda i: (0, 0)),
        ],
        out_specs=pl.BlockSpec((TB, DIM, B), lambda i: (i, 0, 0)),
        out_shape=jax.ShapeDtypeStruct((MEM_TOKENS, DIM, B), jnp.float32),
    )(rows, W_proj.T)

    return outT.transpose(2, 0, 1)


# speedup vs baseline: 1.3211x; 1.0083x over previous
import jax
import jax.numpy as jnp
from jax import lax
from jax.experimental import pallas as pl
from jax.experimental.pallas import tpu as pltpu
from jax.experimental.pallas import tpu_sc as plsc

MEM_TOKENS = 50
DIM = 64
GRP = 8

NC = 2
NS = 16
NW = NC * NS

CHUNK = MEM_TOKENS


def _sc_gather(idx_hbm, table_hbm, out_hbm, idx_v, obuf, dsem, wsem):
    n = idx_hbm.shape[0]
    per_w = n // NW
    nchunk = per_w // CHUNK
    wid = lax.axis_index("s") * NC + lax.axis_index("c")
    base = wid * per_w
    pltpu.sync_copy(idx_hbm.at[pl.ds(base, per_w)], idx_v)

    iota16 = lax.iota(jnp.int32, 16)
    zeros16 = jnp.zeros((16,), jnp.float32)
    wins = [(0, range(16)), (16, range(16)), (32, range(16)),
            (34, range(14, 16))]

    def fire(c, b):
        for off, lanes in wins:
            idx16 = idx_v[pl.ds(c * CHUNK + off, 16)]
            for l in lanes:
                s = idx16[l]
                gi = lax.shift_right_logical(s, 3)
                ri = lax.bitwise_and(s, 7)
                pltpu.make_async_copy(
                    table_hbm.at[gi, pl.ds(ri, 1)],
                    obuf.at[b, off + l],
                    dsem.at[b]).start()

    def drain(b):
        pltpu.make_async_copy(
            out_hbm.at[pl.ds(0, CHUNK), 0, pl.ds(0, 1)], obuf.at[b],
            dsem.at[b]).wait()

    def mask_pass(c, b):
        nz = jnp.zeros((16,), jnp.int32)
        for off, lanes in wins:
            idx16 = idx_v[pl.ds(c * CHUNK + off, 16)]
            if len(lanes) < 16:
                idx16 = jnp.where(iota16 >= lanes[0], idx16, 1)
            nz = nz + jnp.where(idx16 == 0, 1, 0)

        @pl.when(jnp.sum(nz) > 0)
        def _():
            zrow = jnp.zeros((16,), jnp.int32)
            for off, lanes in wins:
                idx16 = idx_v[pl.ds(c * CHUNK + off, 16)]
                z = idx16 == 0
                if len(lanes) < 16:
                    z = z & (iota16 >= lanes[0])
                tok16 = iota16 + off
                for c0 in range(DIM):
                    col = jnp.full((16,), c0, jnp.int32)
                    plsc.store_scatter(obuf.at[b], [tok16, zrow, col],
                                       zeros16, mask=z)

    def write(c, b):
        bb = base // CHUNK + c
        b8 = lax.shift_right_logical(bb, 3)
        br = lax.bitwise_and(bb, 7)
        return pltpu.make_async_copy(
            obuf.at[b],
            out_hbm.at[:, b8, pl.ds(br, 1)],
            wsem.at[b])

    fire(0, 0)

    def body(i, _):
        for b in range(2):
            c = i * 2 + b
            nb = 1 - b

            @pl.when(c + 1 < nchunk)
            def _():
                @pl.when(c >= 1)
                def _():
                    write(c - 1, nb).wait()
                fire(c + 1, nb)

            drain(b)
            mask_pass(c, b)
            write(c, b).start()
        return 0

    lax.fori_loop(0, nchunk // 2, body, 0)

    write(nchunk - 2, 0).wait()
    write(nchunk - 1, 1).wait()


def _tc_matmul(h_ref, wt_ref, o_ref):
    for t in range(h_ref.shape[0]):
        h = h_ref[t]
        acc = jnp.dot(h, wt_ref[...], preferred_element_type=jnp.float32)
        o_ref[t] = acc.T


def kernel(x, emb_table, W_proj):
    B, L = x.shape
    n = B * MEM_TOKENS
    idx = x[:, -MEM_TOKENS:].reshape(n).astype(jnp.int32)
    table3 = emb_table.reshape(emb_table.shape[0] // GRP, GRP, DIM)

    mesh = plsc.VectorSubcoreMesh(core_axis_name="c", subcore_axis_name="s")
    per_w = n // NW
    gather = pl.kernel(
        _sc_gather,
        mesh=mesh,
        out_type=jax.ShapeDtypeStruct((MEM_TOKENS, B // GRP, GRP, DIM),
                                      jnp.float32),
        scratch_types=[
            pltpu.VMEM((per_w,), jnp.int32),
            pltpu.VMEM((2, CHUNK, 1, DIM), jnp.float32),
            pltpu.SemaphoreType.DMA((2,)),
            pltpu.SemaphoreType.DMA((2,)),
        ],
        compiler_params=pltpu.CompilerParams(needs_layout_passes=False),
    )
    rows = gather(idx, table3)
    rows = rows.reshape(MEM_TOKENS, B, DIM)

    TB = 5
    outT = pl.pallas_call(
        _tc_matmul,
        grid=(MEM_TOKENS // TB,),
        in_specs=[
            pl.BlockSpec((TB, B, DIM), lambda i: (i, 0, 0)),
            pl.BlockSpec((DIM, DIM), lambda i: (0, 0)),
        ],
        out_specs=pl.BlockSpec((TB, DIM, B), lambda i: (i, 0, 0)),
        out_shape=jax.ShapeDtypeStruct((MEM_TOKENS, DIM, B), jnp.float32),
    )(rows, W_proj.T)

    return outT.transpose(2, 0, 1)
